# Initial kernel scaffold; baseline (speedup 1.0000x reference)
import jax
import jax.numpy as jnp
from jax.experimental import pallas as pl


def kernel(x, edge_index0, enc_params, net_params, dec_params):
    raise NotImplementedError("write your pallas kernel here")


# speedup vs baseline: 3.7639x; 3.7639x over previous
import functools

import jax
import jax.numpy as jnp
from jax import lax
from jax.experimental import pallas as pl
from jax.experimental.pallas import tpu as pltpu
from jax.experimental.pallas import tpu_sc as plsc

_BN_EPS = 1e-5
_NS = 16
_KE = 128


def _segsum_sc(g2, src2, dst2, zeros_acc):
    n_chunks_total = src2.shape[0]
    nch = n_chunks_total // _NS
    acc_rows = zeros_acc.shape[0]
    rpz = acc_rows // _NS

    mesh = plsc.VectorSubcoreMesh(core_axis_name="c", subcore_axis_name="s")

    @functools.partial(
        pl.kernel,
        mesh=mesh,
        out_type=jax.ShapeDtypeStruct((2, acc_rows, 128), jnp.float32),
        scratch_types=[
            pltpu.VMEM((nch, _KE), jnp.int32),
            pltpu.VMEM((nch, _KE), jnp.int32),
            pltpu.VMEM((_KE, 128), jnp.float32),
            pltpu.VMEM_SHARED((acc_rows, 128), jnp.float32),
            pltpu.SemaphoreType.DMA,
        ],
    )
    def k(g2_h, src_h, dst_h, zero_h, out_h, sidx, didx, rows, acc, sem):
        cid = lax.axis_index("c")
        sid = lax.axis_index("s")
        pltpu.sync_copy(src_h.at[pl.ds(sid * nch, nch)], sidx)
        pltpu.sync_copy(dst_h.at[pl.ds(sid * nch, nch)], didx)
        pltpu.sync_copy(zero_h.at[pl.ds(sid * rpz, rpz)],
                        acc.at[pl.ds(sid * rpz, rpz)])
        plsc.subcore_barrier()

        @pl.loop(0, nch)
        def _(j):
            pltpu.async_copy(g2_h.at[cid].at[sidx.at[j]], rows, sem).wait()
            pltpu.sync_copy(rows, acc.at[didx.at[j]], add=True)

        plsc.subcore_barrier()
        pltpu.sync_copy(acc.at[pl.ds(sid * rpz, rpz)],
                        out_h.at[cid].at[pl.ds(sid * rpz, rpz)])

    return k(g2, src2, dst2, zeros_acc)


_SEGSUM = _segsum_sc


def _stage_a_body(dup_in, g_ref, a_ref, w_ref, b_ref, t1_ref, st_ref):
    z = jnp.concatenate([g_ref[0] + a_ref[0], g_ref[1] + a_ref[1]], axis=1)
    if dup_in:
        z = jnp.concatenate([z, z], axis=1)
    t = jnp.dot(z, w_ref[...], preferred_element_type=jnp.float32)
    t = t + b_ref[...]
    t1_ref[...] = t
    s = jnp.sum(t, axis=0, keepdims=True)
    ss = jnp.sum(t * t, axis=0, keepdims=True)
    st_ref[0] = jnp.concatenate(
        [s, ss, jnp.zeros((6, t.shape[1]), jnp.float32)], axis=0)


def _finalize_bn(st, ga, be, n_rows):
    ssum = jnp.sum(st[:, 0:1, :], axis=0)
    ssq = jnp.sum(st[:, 1:2, :], axis=0)
    mean = ssum / n_rows
    var = ssq / n_rows - mean * mean
    scale = ga * lax.rsqrt(var + _BN_EPS)
    shift = be - mean * scale
    return scale, shift


def _stage_b_body(n_rows, t1_ref, st_ref, ga_ref, be_ref, w1_ref, b1_ref,
                  t2_ref, st2_ref):
    scale, shift = _finalize_bn(st_ref[...], ga_ref[...], be_ref[...], n_rows)
    u = jnp.maximum(t1_ref[...] * scale + shift, 0.0)
    t2 = jnp.dot(u, w1_ref[...], preferred_element_type=jnp.float32)
    t2 = t2 + b1_ref[...]
    t2_ref[...] = t2
    s = jnp.sum(t2, axis=0, keepdims=True)
    ss = jnp.sum(t2 * t2, axis=0, keepdims=True)
    st2_ref[0] = jnp.concatenate(
        [s, ss, jnp.zeros((6, t2.shape[1]), jnp.float32)], axis=0)


def _stage_c_split_body(n_rows, t2_ref, st_ref, ga_ref, be_ref, out_ref):
    scale, shift = _finalize_bn(st_ref[...], ga_ref[...], be_ref[...], n_rows)
    y = jnp.maximum(t2_ref[...] * scale + shift, 0.0)
    out_ref[0] = y[:, :128]
    out_ref[1] = y[:, 128:]


def _stage_c_skip_body(n_rows, t2_ref, st_ref, ga_ref, be_ref, skip_ref,
                       out_ref):
    scale, shift = _finalize_bn(st_ref[...], ga_ref[...], be_ref[...], n_rows)
    y = jnp.maximum(t2_ref[...] * scale + shift, 0.0)
    out_ref[0] = skip_ref[0] + y[:, :128]
    out_ref[1] = skip_ref[1] + y[:, 128:]


def _stage_c_flat_body(n_rows, t2_ref, st_ref, ga_ref, be_ref, out_ref):
    scale, shift = _finalize_bn(st_ref[...], ga_ref[...], be_ref[...], n_rows)
    out_ref[...] = jnp.maximum(t2_ref[...] * scale + shift, 0.0)


def _mlp_tc(g2, agg2, params, skip2, flat_out):
    w0, b0, ga0, be0, w1, b1, ga1, be1 = params
    n = g2.shape[1]
    nb = 5
    rb = n // nb
    din = w0.shape[0]
    h = w0.shape[1]
    dout = w1.shape[1]
    dup_in = din == 512

    row2 = lambda v: v.reshape(1, -1)
    st_shape = jax.ShapeDtypeStruct((nb, 8, h), jnp.float32)
    full2 = lambda r, c: pl.BlockSpec((r, c), lambda i: (0, 0))
    st_spec = pl.BlockSpec((nb, 8, h), lambda i: (0, 0, 0))

    t1, st1 = pl.pallas_call(
        functools.partial(_stage_a_body, dup_in),
        grid=(nb,),
        in_specs=[
            pl.BlockSpec((2, rb, 128), lambda i: (0, i, 0)),
            pl.BlockSpec((2, rb, 128), lambda i: (0, i, 0)),
            full2(din, h), full2(1, h),
        ],
        out_specs=[
            pl.BlockSpec((rb, h), lambda i: (i, 0)),
            pl.BlockSpec((1, 8, h), lambda i: (i, 0, 0)),
        ],
        out_shape=[jax.ShapeDtypeStruct((n, h), jnp.float32), st_shape],
    )(g2, agg2, w0, row2(b0))

    st2_shape = jax.ShapeDtypeStruct((nb, 8, dout), jnp.float32)
    t2, st2 = pl.pallas_call(
        functools.partial(_stage_b_body, float(n)),
        grid=(nb,),
        in_specs=[
            pl.BlockSpec((rb, h), lambda i: (i, 0)),
            st_spec, full2(1, h), full2(1, h),
            full2(h, dout), full2(1, dout),
        ],
        out_specs=[
            pl.BlockSpec((rb, dout), lambda i: (i, 0)),
            pl.BlockSpec((1, 8, dout), lambda i: (i, 0, 0)),
        ],
        out_shape=[jax.ShapeDtypeStruct((n, dout), jnp.float32), st2_shape],
    )(t1, st1, row2(ga0), row2(be0), w1, b1.reshape(1, -1))

    st2_full = pl.BlockSpec((nb, 8, dout), lambda i: (0, 0, 0))
    c_in = [
        pl.BlockSpec((rb, dout), lambda i: (i, 0)),
        st2_full, full2(1, dout), full2(1, dout),
    ]
    c_args = [t2, st2, row2(ga1), row2(be1)]
    if flat_out:
        return pl.pallas_call(
            functools.partial(_stage_c_flat_body, float(n)),
            grid=(nb,),
            in_specs=c_in,
            out_specs=pl.BlockSpec((rb, dout), lambda i: (i, 0)),
            out_shape=jax.ShapeDtypeStruct((n, dout), jnp.float32),
        )(*c_args)
    split_out = pl.BlockSpec((2, rb, 128), lambda i: (0, i, 0))
    out_shape = jax.ShapeDtypeStruct((2, n, 128), jnp.float32)
    if skip2 is None:
        return pl.pallas_call(
            functools.partial(_stage_c_split_body, float(n)),
            grid=(nb,),
            in_specs=c_in,
            out_specs=split_out,
            out_shape=out_shape,
        )(*c_args)
    return pl.pallas_call(
        functools.partial(_stage_c_skip_body, float(n)),
        grid=(nb,),
        in_specs=c_in + [pl.BlockSpec((2, rb, 128), lambda i: (0, i, 0))],
        out_specs=split_out,
        out_shape=out_shape,
    )(*c_args, skip2)


def kernel(x, edge_index0, enc_params, net_params, dec_params):
    n, d = x.shape
    e = edge_index0.shape[1]
    src = edge_index0[0]
    dst = edge_index0[1]

    chunk = _NS * _KE * 8
    e_pad = ((e + chunk - 1) // chunk) * chunk
    acc_rows = ((n + _NS * 8) // (_NS * 8)) * _NS * 8
    src_p = jnp.concatenate([src, jnp.zeros((e_pad - e,), jnp.int32)])
    dst_p = jnp.concatenate([dst, jnp.full((e_pad - e,), acc_rows - 1, jnp.int32)])
    src2 = src_p.reshape(-1, _KE)
    dst2 = dst_p.reshape(-1, _KE)
    zeros_acc = jnp.zeros((acc_rows, 128), jnp.float32)

    g = x.reshape(n, 2, 128).transpose(1, 0, 2)

    agg = _SEGSUM(g, src2, dst2, zeros_acc)
    g = _mlp_tc(g, agg, enc_params, None, False)
    g_skip = g
    for p in net_params:
        agg = _SEGSUM(g, src2, dst2, zeros_acc)
        g = _mlp_tc(g, agg, p, g_skip, False)
    agg = _SEGSUM(g, src2, dst2, zeros_acc)
    return _mlp_tc(g, agg, dec_params, None, True)
